# baseline (speedup 1.0000x reference)
import functools
import jax
import jax.numpy as jnp
from jax import lax
from jax.experimental import pallas as pl
from jax.experimental.pallas import tpu as pltpu
from jax.experimental.pallas import tpu_sc as plsc

_N = 10000
_E = 320000
_D = 128
_P = 3

_NC = 2
_NS = 16
_CH = 32
_DT = 144
_NACC = 10016
_NW = _NC * _NS
_K0 = 300
_K1 = 328
_KMAX = _K1
_E0 = _NS * _K0 * _CH
_ROWS_PER_SUB = _NACC // _NS
_ZFULL = _ROWS_PER_SUB // _CH
_ZREM = _ROWS_PER_SUB - _ZFULL * _CH


def _sc_segment_sum(table, dst3, src3, zeros_blk):
    mesh = plsc.VectorSubcoreMesh(core_axis_name="c", subcore_axis_name="s")

    @functools.partial(
        pl.kernel,
        out_type=jax.ShapeDtypeStruct((_NC, _NACC, _DT), jnp.float32),
        mesh=mesh,
        scratch_types=[
            pltpu.VMEM((_KMAX, _CH), jnp.int32),
            pltpu.VMEM((_KMAX, _CH), jnp.int32),
            pltpu.VMEM((_CH, _DT), jnp.float32),
            pltpu.VMEM((_CH, _DT), jnp.float32),
            pltpu.VMEM((_CH, _DT), jnp.float32),
            pltpu.VMEM((_CH, _DT), jnp.float32),
            pltpu.SemaphoreType.DMA,
            pltpu.SemaphoreType.DMA,
            pltpu.SemaphoreType.DMA,
            pltpu.SemaphoreType.DMA,
            pltpu.SemaphoreType.DMA,
            pltpu.SemaphoreType.DMA,
            pltpu.SemaphoreType.DMA,
            pltpu.SemaphoreType.DMA,
            pltpu.VMEM_SHARED((_NACC, _DT), jnp.float32),
        ],
        compiler_params=pltpu.CompilerParams(use_tc_tiling_on_sc=False),
    )
    def k(table_hbm, dst_hbm, src_hbm, zero_hbm, out_hbm,
          dst_v, src_v, rows_a, rows_b, rows_c, rows_d,
          gs0, gs1, gs2, gs3, ss0, ss1, ss2, ss3, acc):
        c = lax.axis_index("c")
        s = lax.axis_index("s")
        wid = c * _NS + s
        k_me = jnp.where(c == 0, _K0, _K1)
        bufs = (rows_a, rows_b, rows_c, rows_d)
        gsems = (gs0, gs1, gs2, gs3)
        ssems = (ss0, ss1, ss2, ss3)

        pltpu.sync_copy(zero_hbm, rows_a)
        base = s * _ROWS_PER_SUB
        for i in range(_ZFULL):
            pltpu.sync_copy(rows_a, acc.at[pl.ds(base + i * _CH, _CH)])
        if _ZREM:
            pltpu.sync_copy(rows_a.at[pl.ds(0, _ZREM)],
                            acc.at[pl.ds(base + _ZFULL * _CH, _ZREM)])

        pltpu.sync_copy(dst_hbm.at[wid], dst_v)
        pltpu.sync_copy(src_hbm.at[wid], src_v)
        plsc.subcore_barrier()

        def gath(j, b):
            return pltpu.make_async_copy(
                table_hbm.at[dst_v.at[j]], bufs[b], gsems[b])

        def scat(j, b):
            return pltpu.make_async_copy(
                bufs[b], acc.at[src_v.at[j]], ssems[b])

        gath(0, 0).start()

        def body(g, carry):
            for b in range(4):
                j = 4 * g + b
                gath(j, b).wait()
                pltpu.async_copy(bufs[b], acc.at[src_v.at[j]], ssems[b],
                                 add=True)
                nb = (b + 1) % 4

                @pl.when(j >= 3)
                def _():
                    scat(j - 3, nb).wait()

                @pl.when(j + 1 < k_me)
                def _():
                    gath(j + 1, nb).start()
            return carry

        lax.fori_loop(0, k_me // 4, body, 0)
        scat(k_me - 3, 1).wait()
        scat(k_me - 2, 2).wait()
        scat(k_me - 1, 3).wait()

        plsc.subcore_barrier()
        pltpu.sync_copy(acc.at[pl.ds(s * _ROWS_PER_SUB, _ROWS_PER_SUB)],
                        out_hbm.at[c, pl.ds(s * _ROWS_PER_SUB, _ROWS_PER_SUB)])

    return k(table, dst3, src3, zeros_blk)


_BN = 1024


def _tc_body(part_ref, x_ref, pos_ref, w1_ref, w2_ref, w3_ref, wcat_ref,
             wa1_ref, wa2_ref, bm_ref, ba_ref, out_ref):
    S = part_ref[0] + part_ref[1]
    cnt = S[:, _D + _P:_D + _P + 1]
    inv = 1.0 / jnp.maximum(cnt, 1.0)
    has = (cnt > 0.0).astype(jnp.float32)
    M = S * inv
    xb = x_ref[...]
    self_term = (jnp.dot(xb, w1_ref[...] - w2_ref[...],
                         preferred_element_type=jnp.float32)
                 - jnp.dot(pos_ref[...], w3_ref[...],
                           preferred_element_type=jnp.float32)
                 + bm_ref[...])
    aggr = has * self_term + jnp.dot(M, wcat_ref[...],
                                     preferred_element_type=jnp.float32)
    out_ref[...] = (jnp.dot(xb, wa1_ref[...], preferred_element_type=jnp.float32)
                    + jnp.dot(aggr, wa2_ref[...], preferred_element_type=jnp.float32)
                    + ba_ref[...])


def _tc_update(part, x_p, pos_p, w1, w2, w3p, wcat, wa1, wa2, bm, ba):
    grid = (pl.cdiv(_NACC, _BN),)
    full = lambda shape: pl.BlockSpec(shape, lambda i: (0,) * len(shape))
    return pl.pallas_call(
        _tc_body,
        grid=grid,
        in_specs=[
            pl.BlockSpec((_NC, _BN, _DT), lambda i: (0, i, 0)),
            pl.BlockSpec((_BN, _D), lambda i: (i, 0)),
            pl.BlockSpec((_BN, 8), lambda i: (i, 0)),
            full((_D, _D)), full((_D, _D)), full((8, _D)), full((_DT, _D)),
            full((_D, _D)), full((_D, _D)), full((1, _D)), full((1, _D)),
        ],
        out_specs=pl.BlockSpec((_BN, _D), lambda i: (i, 0)),
        out_shape=jax.ShapeDtypeStruct((_NACC, _D), jnp.float32),
    )(part, x_p, pos_p, w1, w2, w3p, wcat, wa1, wa2, bm, ba)


def kernel(x, edge_index, pos, W_msg, b_msg, W_agg, b_agg):
    src = edge_index[0].astype(jnp.int32)
    dst = edge_index[1].astype(jnp.int32)

    table = jnp.concatenate(
        [x, pos, jnp.ones((_N, 1), jnp.float32),
         jnp.zeros((_N, _DT - _D - _P - 1), jnp.float32)], axis=1)

    def _layout(idx, fill):
        pad0 = _NS * (_KMAX - _K0) * _CH
        pad1 = _NS * _K1 * _CH - (_E - _E0)
        half0 = jnp.concatenate(
            [idx[:_E0].reshape(_NS, _K0, _CH),
             jnp.full((pad0,), fill, jnp.int32).reshape(_NS, _KMAX - _K0, _CH)],
            axis=1)
        half1 = jnp.concatenate(
            [idx[_E0:], jnp.full((pad1,), fill, jnp.int32)]).reshape(
            _NS, _K1, _CH)
        return jnp.concatenate([half0, half1], axis=0)

    src3 = _layout(src, _N)
    dst3 = _layout(dst, 0)
    zeros_blk = jnp.zeros((_CH, _DT), jnp.float32)

    part = _sc_segment_sum(table, dst3, src3, zeros_blk)

    x_p = jnp.concatenate([x, jnp.zeros((_NACC - _N, _D), jnp.float32)])
    pos_p = jnp.concatenate(
        [jnp.concatenate([pos, jnp.zeros((_N, 8 - _P), jnp.float32)], axis=1),
         jnp.zeros((_NACC - _N, 8), jnp.float32)])
    w1 = W_msg[:_D]
    w2 = W_msg[_D:2 * _D]
    w3p = jnp.concatenate([W_msg[2 * _D:], jnp.zeros((8 - _P, _D), jnp.float32)])
    wcat = jnp.concatenate(
        [w2, W_msg[2 * _D:], jnp.zeros((_DT - 2 * _D - _P + _D, _D), jnp.float32)])
    wa1 = W_agg[:_D]
    wa2 = W_agg[_D:]
    bm = b_msg.reshape(1, _D)
    ba = b_agg.reshape(1, _D)

    out = _tc_update(part, x_p, pos_p, w1, w2, w3p, wcat, wa1, wa2, bm, ba)
    return out[:_N]


# speedup vs baseline: 1.1620x; 1.1620x over previous
import functools
import jax
import jax.numpy as jnp
from jax import lax
from jax.experimental import pallas as pl
from jax.experimental.pallas import tpu as pltpu
from jax.experimental.pallas import tpu_sc as plsc

_N = 10000
_E = 320000
_D = 128
_P = 3

_NC = 2
_NS = 16
_CH = 48
_DT = 144
_NACC = 10016
_NW = _NC * _NS
_K0 = 200
_K1 = 218
_KMAX = _K1
_E0 = _NS * _K0 * _CH
_ROWS_PER_SUB = _NACC // _NS
_ZFULL = _ROWS_PER_SUB // _CH
_ZREM = _ROWS_PER_SUB - _ZFULL * _CH


def _sc_segment_sum(table, dst3, src3, zeros_blk):
    mesh = plsc.VectorSubcoreMesh(core_axis_name="c", subcore_axis_name="s")

    @functools.partial(
        pl.kernel,
        out_type=jax.ShapeDtypeStruct((_NC, _NACC, _DT), jnp.float32),
        mesh=mesh,
        scratch_types=[
            pltpu.VMEM((_KMAX, _CH), jnp.int32),
            pltpu.VMEM((_KMAX, _CH), jnp.int32),
            pltpu.VMEM((_CH, _DT), jnp.float32),
            pltpu.VMEM((_CH, _DT), jnp.float32),
            pltpu.SemaphoreType.DMA,
            pltpu.SemaphoreType.DMA,
            pltpu.VMEM_SHARED((_NACC, _DT), jnp.float32),
        ],
        compiler_params=pltpu.CompilerParams(use_tc_tiling_on_sc=False),
    )
    def k(table_hbm, dst_hbm, src_hbm, zero_hbm, out_hbm,
          dst_v, src_v, rows_a, rows_b, sem_a, sem_b, acc):
        c = lax.axis_index("c")
        s = lax.axis_index("s")
        wid = c * _NS + s
        k_me = jnp.where(c == 0, _K0, _K1)
        bufs = (rows_a, rows_b)
        sems = (sem_a, sem_b)

        pltpu.sync_copy(zero_hbm, rows_a)
        base = s * _ROWS_PER_SUB
        for i in range(_ZFULL):
            pltpu.sync_copy(rows_a, acc.at[pl.ds(base + i * _CH, _CH)])
        if _ZREM:
            pltpu.sync_copy(rows_a.at[pl.ds(0, _ZREM)],
                            acc.at[pl.ds(base + _ZFULL * _CH, _ZREM)])

        pltpu.sync_copy(dst_hbm.at[wid], dst_v)
        pltpu.sync_copy(src_hbm.at[wid], src_v)
        plsc.subcore_barrier()

        pltpu.make_async_copy(table_hbm.at[dst_v.at[0]], rows_a, sem_a).start()

        def body(g, carry):
            for b in range(2):
                j = 2 * g + b
                pltpu.make_async_copy(
                    table_hbm.at[dst_v.at[j]], bufs[b], sems[b]).wait()

                @pl.when(j + 1 < k_me)
                def _():
                    pltpu.make_async_copy(
                        table_hbm.at[dst_v.at[j + 1]],
                        bufs[1 - b], sems[1 - b]).start()

                pltpu.sync_copy(bufs[b], acc.at[src_v.at[j]], add=True)
            return carry

        lax.fori_loop(0, k_me // 2, body, 0)

        plsc.subcore_barrier()
        pltpu.sync_copy(acc.at[pl.ds(s * _ROWS_PER_SUB, _ROWS_PER_SUB)],
                        out_hbm.at[c, pl.ds(s * _ROWS_PER_SUB, _ROWS_PER_SUB)])

    return k(table, dst3, src3, zeros_blk)


_BN = 1024


def _tc_body(part_ref, x_ref, pos_ref, w1_ref, w2_ref, w3_ref, wcat_ref,
             wa1_ref, wa2_ref, bm_ref, ba_ref, out_ref):
    S = part_ref[0] + part_ref[1]
    cnt = S[:, _D + _P:_D + _P + 1]
    inv = 1.0 / jnp.maximum(cnt, 1.0)
    has = (cnt > 0.0).astype(jnp.float32)
    M = S * inv
    xb = x_ref[...]
    self_term = (jnp.dot(xb, w1_ref[...] - w2_ref[...],
                         preferred_element_type=jnp.float32)
                 - jnp.dot(pos_ref[...], w3_ref[...],
                           preferred_element_type=jnp.float32)
                 + bm_ref[...])
    aggr = has * self_term + jnp.dot(M, wcat_ref[...],
                                     preferred_element_type=jnp.float32)
    out_ref[...] = (jnp.dot(xb, wa1_ref[...], preferred_element_type=jnp.float32)
                    + jnp.dot(aggr, wa2_ref[...], preferred_element_type=jnp.float32)
                    + ba_ref[...])


def _tc_update(part, x_p, pos_p, w1, w2, w3p, wcat, wa1, wa2, bm, ba):
    grid = (pl.cdiv(_NACC, _BN),)
    full = lambda shape: pl.BlockSpec(shape, lambda i: (0,) * len(shape))
    return pl.pallas_call(
        _tc_body,
        grid=grid,
        in_specs=[
            pl.BlockSpec((_NC, _BN, _DT), lambda i: (0, i, 0)),
            pl.BlockSpec((_BN, _D), lambda i: (i, 0)),
            pl.BlockSpec((_BN, 8), lambda i: (i, 0)),
            full((_D, _D)), full((_D, _D)), full((8, _D)), full((_DT, _D)),
            full((_D, _D)), full((_D, _D)), full((1, _D)), full((1, _D)),
        ],
        out_specs=pl.BlockSpec((_BN, _D), lambda i: (i, 0)),
        out_shape=jax.ShapeDtypeStruct((_NACC, _D), jnp.float32),
    )(part, x_p, pos_p, w1, w2, w3p, wcat, wa1, wa2, bm, ba)


def kernel(x, edge_index, pos, W_msg, b_msg, W_agg, b_agg):
    src = edge_index[0].astype(jnp.int32)
    dst = edge_index[1].astype(jnp.int32)

    table = jnp.concatenate(
        [x, pos, jnp.ones((_N, 1), jnp.float32),
         jnp.zeros((_N, _DT - _D - _P - 1), jnp.float32)], axis=1)

    def _layout(idx, fill):
        pad0 = _NS * (_KMAX - _K0) * _CH
        pad1 = _NS * _K1 * _CH - (_E - _E0)
        half0 = jnp.concatenate(
            [idx[:_E0].reshape(_NS, _K0, _CH),
             jnp.full((pad0,), fill, jnp.int32).reshape(_NS, _KMAX - _K0, _CH)],
            axis=1)
        half1 = jnp.concatenate(
            [idx[_E0:], jnp.full((pad1,), fill, jnp.int32)]).reshape(
            _NS, _K1, _CH)
        return jnp.concatenate([half0, half1], axis=0)

    src3 = _layout(src, _N)
    dst3 = _layout(dst, 0)
    zeros_blk = jnp.zeros((_CH, _DT), jnp.float32)

    part = _sc_segment_sum(table, dst3, src3, zeros_blk)

    x_p = jnp.concatenate([x, jnp.zeros((_NACC - _N, _D), jnp.float32)])
    pos_p = jnp.concatenate(
        [jnp.concatenate([pos, jnp.zeros((_N, 8 - _P), jnp.float32)], axis=1),
         jnp.zeros((_NACC - _N, 8), jnp.float32)])
    w1 = W_msg[:_D]
    w2 = W_msg[_D:2 * _D]
    w3p = jnp.concatenate([W_msg[2 * _D:], jnp.zeros((8 - _P, _D), jnp.float32)])
    wcat = jnp.concatenate(
        [w2, W_msg[2 * _D:], jnp.zeros((_DT - 2 * _D - _P + _D, _D), jnp.float32)])
    wa1 = W_agg[:_D]
    wa2 = W_agg[_D:]
    bm = b_msg.reshape(1, _D)
    ba = b_agg.reshape(1, _D)

    out = _tc_update(part, x_p, pos_p, w1, w2, w3p, wcat, wa1, wa2, bm, ba)
    return out[:_N]


# speedup vs baseline: 1.2459x; 1.0722x over previous
import functools
import jax
import jax.numpy as jnp
from jax import lax
from jax.experimental import pallas as pl
from jax.experimental.pallas import tpu as pltpu
from jax.experimental.pallas import tpu_sc as plsc

_N = 10000
_E = 320000
_D = 128
_P = 3

_NC = 2
_NS = 16
_CH = 48
_DT = 144
_NACC = 10016
_NW = _NC * _NS
_K0 = 224
_K1 = 194
_KMAX = max(_K0, _K1)
_E0 = _NS * _K0 * _CH
_ROWS_PER_SUB = _NACC // _NS
_ZFULL = _ROWS_PER_SUB // _CH
_ZREM = _ROWS_PER_SUB - _ZFULL * _CH


def _sc_segment_sum(table, dst3, src3, zeros_blk):
    mesh = plsc.VectorSubcoreMesh(core_axis_name="c", subcore_axis_name="s")

    @functools.partial(
        pl.kernel,
        out_type=jax.ShapeDtypeStruct((_NC, _NACC, _DT), jnp.float32),
        mesh=mesh,
        scratch_types=[
            pltpu.VMEM((_KMAX, _CH), jnp.int32),
            pltpu.VMEM((_KMAX, _CH), jnp.int32),
            pltpu.VMEM((_CH, _DT), jnp.float32),
            pltpu.VMEM((_CH, _DT), jnp.float32),
            pltpu.SemaphoreType.DMA,
            pltpu.SemaphoreType.DMA,
            pltpu.VMEM_SHARED((_NACC, _DT), jnp.float32),
        ],
        compiler_params=pltpu.CompilerParams(use_tc_tiling_on_sc=False),
    )
    def k(table_hbm, dst_hbm, src_hbm, zero_hbm, out_hbm,
          dst_v, src_v, rows_a, rows_b, sem_a, sem_b, acc):
        c = lax.axis_index("c")
        s = lax.axis_index("s")
        wid = c * _NS + s
        k_me = jnp.where(c == 0, _K0, _K1)
        bufs = (rows_a, rows_b)
        sems = (sem_a, sem_b)

        pltpu.sync_copy(zero_hbm, rows_a)
        base = s * _ROWS_PER_SUB
        for i in range(_ZFULL):
            pltpu.sync_copy(rows_a, acc.at[pl.ds(base + i * _CH, _CH)])
        if _ZREM:
            pltpu.sync_copy(rows_a.at[pl.ds(0, _ZREM)],
                            acc.at[pl.ds(base + _ZFULL * _CH, _ZREM)])

        pltpu.sync_copy(dst_hbm.at[wid], dst_v)
        pltpu.sync_copy(src_hbm.at[wid], src_v)
        plsc.subcore_barrier()

        pltpu.make_async_copy(table_hbm.at[dst_v.at[0]], rows_a, sem_a).start()

        def body(g, carry):
            for b in range(2):
                j = 2 * g + b
                pltpu.make_async_copy(
                    table_hbm.at[dst_v.at[j]], bufs[b], sems[b]).wait()

                @pl.when(j + 1 < k_me)
                def _():
                    pltpu.make_async_copy(
                        table_hbm.at[dst_v.at[j + 1]],
                        bufs[1 - b], sems[1 - b]).start()

                pltpu.sync_copy(bufs[b], acc.at[src_v.at[j]], add=True)
            return carry

        lax.fori_loop(0, k_me // 2, body, 0)

        plsc.subcore_barrier()
        pltpu.sync_copy(acc.at[pl.ds(s * _ROWS_PER_SUB, _ROWS_PER_SUB)],
                        out_hbm.at[c, pl.ds(s * _ROWS_PER_SUB, _ROWS_PER_SUB)])

    return k(table, dst3, src3, zeros_blk)


_BN = 1024


def _tc_body(part_ref, x_ref, pos_ref, w1_ref, w2_ref, w3_ref, wcat_ref,
             wa1_ref, wa2_ref, bm_ref, ba_ref, out_ref):
    S = part_ref[0] + part_ref[1]
    cnt = S[:, _D + _P:_D + _P + 1]
    inv = 1.0 / jnp.maximum(cnt, 1.0)
    has = (cnt > 0.0).astype(jnp.float32)
    M = S * inv
    xb = x_ref[...]
    self_term = (jnp.dot(xb, w1_ref[...] - w2_ref[...],
                         preferred_element_type=jnp.float32)
                 - jnp.dot(pos_ref[...], w3_ref[...],
                           preferred_element_type=jnp.float32)
                 + bm_ref[...])
    aggr = has * self_term + jnp.dot(M, wcat_ref[...],
                                     preferred_element_type=jnp.float32)
    out_ref[...] = (jnp.dot(xb, wa1_ref[...], preferred_element_type=jnp.float32)
                    + jnp.dot(aggr, wa2_ref[...], preferred_element_type=jnp.float32)
                    + ba_ref[...])


def _tc_update(part, x_p, pos_p, w1, w2, w3p, wcat, wa1, wa2, bm, ba):
    grid = (pl.cdiv(_NACC, _BN),)
    full = lambda shape: pl.BlockSpec(shape, lambda i: (0,) * len(shape))
    return pl.pallas_call(
        _tc_body,
        grid=grid,
        in_specs=[
            pl.BlockSpec((_NC, _BN, _DT), lambda i: (0, i, 0)),
            pl.BlockSpec((_BN, _D), lambda i: (i, 0)),
            pl.BlockSpec((_BN, 8), lambda i: (i, 0)),
            full((_D, _D)), full((_D, _D)), full((8, _D)), full((_DT, _D)),
            full((_D, _D)), full((_D, _D)), full((1, _D)), full((1, _D)),
        ],
        out_specs=pl.BlockSpec((_BN, _D), lambda i: (i, 0)),
        out_shape=jax.ShapeDtypeStruct((_NACC, _D), jnp.float32),
    )(part, x_p, pos_p, w1, w2, w3p, wcat, wa1, wa2, bm, ba)


def kernel(x, edge_index, pos, W_msg, b_msg, W_agg, b_agg):
    src = edge_index[0].astype(jnp.int32)
    dst = edge_index[1].astype(jnp.int32)

    table = jnp.concatenate(
        [x, pos, jnp.ones((_N, 1), jnp.float32),
         jnp.zeros((_N, _DT - _D - _P - 1), jnp.float32)], axis=1)

    def _layout(idx, fill):
        pad1 = _NS * _K1 * _CH - (_E - _E0)
        half0 = idx[:_E0].reshape(_NS, _K0, _CH)
        half1 = jnp.concatenate(
            [idx[_E0:], jnp.full((pad1,), fill, jnp.int32)]).reshape(
            _NS, _K1, _CH)

        def _to_kmax(h, kh):
            if kh == _KMAX:
                return h
            fill_blk = jnp.full((_NS, _KMAX - kh, _CH), fill, jnp.int32)
            return jnp.concatenate([h, fill_blk], axis=1)

        return jnp.concatenate(
            [_to_kmax(half0, _K0), _to_kmax(half1, _K1)], axis=0)

    src3 = _layout(src, _N)
    dst3 = _layout(dst, 0)
    zeros_blk = jnp.zeros((_CH, _DT), jnp.float32)

    part = _sc_segment_sum(table, dst3, src3, zeros_blk)

    x_p = jnp.concatenate([x, jnp.zeros((_NACC - _N, _D), jnp.float32)])
    pos_p = jnp.concatenate(
        [jnp.concatenate([pos, jnp.zeros((_N, 8 - _P), jnp.float32)], axis=1),
         jnp.zeros((_NACC - _N, 8), jnp.float32)])
    w1 = W_msg[:_D]
    w2 = W_msg[_D:2 * _D]
    w3p = jnp.concatenate([W_msg[2 * _D:], jnp.zeros((8 - _P, _D), jnp.float32)])
    wcat = jnp.concatenate(
        [w2, W_msg[2 * _D:], jnp.zeros((_DT - 2 * _D - _P + _D, _D), jnp.float32)])
    wa1 = W_agg[:_D]
    wa2 = W_agg[_D:]
    bm = b_msg.reshape(1, _D)
    ba = b_agg.reshape(1, _D)

    out = _tc_update(part, x_p, pos_p, w1, w2, w3p, wcat, wa1, wa2, bm, ba)
    return out[:_N]


# speedup vs baseline: 1.2789x; 1.0264x over previous
import functools
import jax
import jax.numpy as jnp
from jax import lax
from jax.experimental import pallas as pl
from jax.experimental.pallas import tpu as pltpu
from jax.experimental.pallas import tpu_sc as plsc

_N = 10000
_E = 320000
_D = 128
_P = 3

_NC = 2
_NS = 16
_CH = 64
_DT = 144
_NACC = 10016
_NW = _NC * _NS
_K0 = 168
_K1 = 146
_KMAX = max(_K0, _K1)
_E0 = _NS * _K0 * _CH
_ROWS_PER_SUB = _NACC // _NS
_ZFULL = _ROWS_PER_SUB // _CH
_ZREM = _ROWS_PER_SUB - _ZFULL * _CH


def _sc_segment_sum(table, dst3, src3, zeros_blk):
    mesh = plsc.VectorSubcoreMesh(core_axis_name="c", subcore_axis_name="s")

    @functools.partial(
        pl.kernel,
        out_type=jax.ShapeDtypeStruct((_NC, _NACC, _DT), jnp.float32),
        mesh=mesh,
        scratch_types=[
            pltpu.VMEM((_KMAX, _CH), jnp.int32),
            pltpu.VMEM((_KMAX, _CH), jnp.int32),
            pltpu.VMEM((_CH, _DT), jnp.float32),
            pltpu.VMEM((_CH, _DT), jnp.float32),
            pltpu.SemaphoreType.DMA,
            pltpu.SemaphoreType.DMA,
            pltpu.VMEM_SHARED((_NACC, _DT), jnp.float32),
        ],
        compiler_params=pltpu.CompilerParams(use_tc_tiling_on_sc=False),
    )
    def k(table_hbm, dst_hbm, src_hbm, zero_hbm, out_hbm,
          dst_v, src_v, rows_a, rows_b, sem_a, sem_b, acc):
        c = lax.axis_index("c")
        s = lax.axis_index("s")
        wid = c * _NS + s
        k_me = jnp.where(c == 0, _K0, _K1)
        bufs = (rows_a, rows_b)
        sems = (sem_a, sem_b)

        pltpu.make_async_copy(dst_hbm.at[wid], dst_v, sem_a).start()
        pltpu.make_async_copy(src_hbm.at[wid], src_v, sem_b).start()
        pltpu.sync_copy(zero_hbm, rows_a)
        base = s * _ROWS_PER_SUB
        zcopies = []
        for i in range(_ZFULL):
            zcopies.append(pltpu.make_async_copy(
                rows_a, acc.at[pl.ds(base + i * _CH, _CH)], sem_a))
        if _ZREM:
            zcopies.append(pltpu.make_async_copy(
                rows_a.at[pl.ds(0, _ZREM)],
                acc.at[pl.ds(base + _ZFULL * _CH, _ZREM)], sem_a))
        pltpu.make_async_copy(dst_hbm.at[wid], dst_v, sem_a).wait()
        for zc in zcopies:
            zc.start()
        for zc in zcopies:
            zc.wait()
        pltpu.make_async_copy(src_hbm.at[wid], src_v, sem_b).wait()
        plsc.subcore_barrier()

        pltpu.make_async_copy(table_hbm.at[dst_v.at[0]], rows_a, sem_a).start()

        def body(g, carry):
            for b in range(2):
                j = 2 * g + b
                pltpu.make_async_copy(
                    table_hbm.at[dst_v.at[j]], bufs[b], sems[b]).wait()

                @pl.when(j + 1 < k_me)
                def _():
                    pltpu.make_async_copy(
                        table_hbm.at[dst_v.at[j + 1]],
                        bufs[1 - b], sems[1 - b]).start()

                pltpu.sync_copy(bufs[b], acc.at[src_v.at[j]], add=True)
            return carry

        lax.fori_loop(0, k_me // 2, body, 0)

        plsc.subcore_barrier()
        pltpu.sync_copy(acc.at[pl.ds(s * _ROWS_PER_SUB, _ROWS_PER_SUB)],
                        out_hbm.at[c, pl.ds(s * _ROWS_PER_SUB, _ROWS_PER_SUB)])

    return k(table, dst3, src3, zeros_blk)


_BN = 1024


def _tc_body(part_ref, x_ref, pos_ref, w1_ref, w2_ref, w3_ref, wcat_ref,
             wa1_ref, wa2_ref, bm_ref, ba_ref, out_ref):
    S = part_ref[0] + part_ref[1]
    cnt = S[:, _D + _P:_D + _P + 1]
    inv = 1.0 / jnp.maximum(cnt, 1.0)
    has = (cnt > 0.0).astype(jnp.float32)
    M = S * inv
    xb = x_ref[...]
    self_term = (jnp.dot(xb, w1_ref[...] - w2_ref[...],
                         preferred_element_type=jnp.float32)
                 - jnp.dot(pos_ref[...], w3_ref[...],
                           preferred_element_type=jnp.float32)
                 + bm_ref[...])
    aggr = has * self_term + jnp.dot(M, wcat_ref[...],
                                     preferred_element_type=jnp.float32)
    out_ref[...] = (jnp.dot(xb, wa1_ref[...], preferred_element_type=jnp.float32)
                    + jnp.dot(aggr, wa2_ref[...], preferred_element_type=jnp.float32)
                    + ba_ref[...])


def _tc_update(part, x_p, pos_p, w1, w2, w3p, wcat, wa1, wa2, bm, ba):
    grid = (pl.cdiv(_NACC, _BN),)
    full = lambda shape: pl.BlockSpec(shape, lambda i: (0,) * len(shape))
    return pl.pallas_call(
        _tc_body,
        grid=grid,
        in_specs=[
            pl.BlockSpec((_NC, _BN, _DT), lambda i: (0, i, 0)),
            pl.BlockSpec((_BN, _D), lambda i: (i, 0)),
            pl.BlockSpec((_BN, 8), lambda i: (i, 0)),
            full((_D, _D)), full((_D, _D)), full((8, _D)), full((_DT, _D)),
            full((_D, _D)), full((_D, _D)), full((1, _D)), full((1, _D)),
        ],
        out_specs=pl.BlockSpec((_BN, _D), lambda i: (i, 0)),
        out_shape=jax.ShapeDtypeStruct((_NACC, _D), jnp.float32),
    )(part, x_p, pos_p, w1, w2, w3p, wcat, wa1, wa2, bm, ba)


def kernel(x, edge_index, pos, W_msg, b_msg, W_agg, b_agg):
    src = edge_index[0].astype(jnp.int32)
    dst = edge_index[1].astype(jnp.int32)

    table = jnp.concatenate(
        [x, pos, jnp.ones((_N, 1), jnp.float32),
         jnp.zeros((_N, _DT - _D - _P - 1), jnp.float32)], axis=1)

    def _layout(idx, fill):
        pad1 = _NS * _K1 * _CH - (_E - _E0)
        half0 = idx[:_E0].reshape(_NS, _K0, _CH)
        half1 = jnp.concatenate(
            [idx[_E0:], jnp.full((pad1,), fill, jnp.int32)]).reshape(
            _NS, _K1, _CH)

        def _to_kmax(h, kh):
            if kh == _KMAX:
                return h
            fill_blk = jnp.full((_NS, _KMAX - kh, _CH), fill, jnp.int32)
            return jnp.concatenate([h, fill_blk], axis=1)

        return jnp.concatenate(
            [_to_kmax(half0, _K0), _to_kmax(half1, _K1)], axis=0)

    src3 = _layout(src, _N)
    dst3 = _layout(dst, 0)
    zeros_blk = jnp.zeros((_CH, _DT), jnp.float32)

    part = _sc_segment_sum(table, dst3, src3, zeros_blk)

    x_p = jnp.concatenate([x, jnp.zeros((_NACC - _N, _D), jnp.float32)])
    pos_p = jnp.concatenate(
        [jnp.concatenate([pos, jnp.zeros((_N, 8 - _P), jnp.float32)], axis=1),
         jnp.zeros((_NACC - _N, 8), jnp.float32)])
    w1 = W_msg[:_D]
    w2 = W_msg[_D:2 * _D]
    w3p = jnp.concatenate([W_msg[2 * _D:], jnp.zeros((8 - _P, _D), jnp.float32)])
    wcat = jnp.concatenate(
        [w2, W_msg[2 * _D:], jnp.zeros((_DT - 2 * _D - _P + _D, _D), jnp.float32)])
    wa1 = W_agg[:_D]
    wa2 = W_agg[_D:]
    bm = b_msg.reshape(1, _D)
    ba = b_agg.reshape(1, _D)

    out = _tc_update(part, x_p, pos_p, w1, w2, w3p, wcat, wa1, wa2, bm, ba)
    return out[:_N]
